# baseline (speedup 1.0000x reference)
import functools

import jax
import jax.numpy as jnp
from jax import lax
from jax.experimental import pallas as pl
from jax.experimental.pallas import tpu as pltpu
from jax.experimental.pallas import tpu_sc as plsc

_D = 32
_G = 4
_NC = 2
_NS = 16
_NW = _NC * _NS
_LANES = 16


def _pack_rows(t_T):
    d, rows = t_T.shape
    blk = 8192
    nblk = pl.cdiv(rows, blk)

    def body(t_ref, o_ref):
        eye = (
            lax.broadcasted_iota(jnp.int32, (d, d), 0)
            == lax.broadcasted_iota(jnp.int32, (d, d), 1)
        ).astype(jnp.float32)
        t1 = lax.dot_general(
            t_ref[...], eye, (((0,), (0,)), ((), ())),
            preferred_element_type=jnp.float32,
            precision=lax.Precision.HIGHEST,
        )
        t1r = t1.reshape(blk // _G, _G, d)
        for q in range(_G):
            o_ref[:, q * d:(q + 1) * d] = t1r[:, q, :]

    return pl.pallas_call(
        body,
        grid=(nblk,),
        in_specs=[pl.BlockSpec((d, blk), lambda g: (0, g))],
        out_specs=pl.BlockSpec((blk // _G, _G * d), lambda g: (g, 0)),
        out_shape=jax.ShapeDtypeStruct((nblk * blk // _G, _G * d), jnp.float32),
    )(t_T)


def _gather_rows(grp, sel, table2):
    total = grp.shape[0]
    per_w = total // _NW
    chunk = 160
    n_chunks = per_w // chunk

    mesh = plsc.VectorSubcoreMesh(core_axis_name="c", subcore_axis_name="s")

    @functools.partial(
        pl.kernel,
        mesh=mesh,
        out_type=jax.ShapeDtypeStruct((total, _D), jnp.float32),
        scratch_types=[
            pltpu.VMEM((per_w,), jnp.int32),
            pltpu.VMEM((per_w,), jnp.int32),
            pltpu.VMEM((chunk, _G * _D), jnp.float32),
            pltpu.VMEM((chunk, _D), jnp.float32),
            pltpu.SemaphoreType.DMA,
            pltpu.SemaphoreType.DMA,
        ],
        compiler_params=pltpu.CompilerParams(
            use_tc_tiling_on_sc=True, needs_layout_passes=False
        ),
    )
    def k(grp_hbm, sel_hbm, table_hbm, out_hbm, grp_v, sel_v, staged, comp,
          gsem, wsem):
        wid = lax.axis_index("s") * _NC + lax.axis_index("c")
        base = wid * per_w
        pltpu.sync_copy(grp_hbm.at[pl.ds(base, per_w)], grp_v)
        pltpu.sync_copy(sel_hbm.at[pl.ds(base, per_w)], sel_v)

        lanes_iota = lax.broadcasted_iota(jnp.int32, (_LANES,), 0)

        def body(j, carry):
            off = j * chunk
            pltpu.async_copy(
                table_hbm.at[grp_v.at[pl.ds(off, chunk)]], staged, gsem
            ).wait()
            for t in range(chunk):
                col0 = plsc.load_gather(
                    sel_v, [jnp.full((_LANES,), off + t, jnp.int32)]
                )
                trow = jnp.full((_LANES,), t, jnp.int32)
                for j0 in (0, _LANES):
                    val = plsc.load_gather(
                        staged, [trow, col0 + (j0 + lanes_iota)]
                    )
                    comp[t, pl.ds(j0, _LANES)] = val
            pltpu.async_copy(
                comp, out_hbm.at[pl.ds(base + off, chunk)], wsem
            ).wait()
            return carry

        lax.fori_loop(0, n_chunks, body, 0)

    return k(grp, sel, table2)


def kernel(user, memory, item_table, user_table):
    b, n = memory.shape
    idx = memory.reshape(b * n).astype(jnp.int32)
    table2 = _pack_rows(item_table.T)
    grp = idx >> 2
    sel = (idx & (_G - 1)) * _D
    out = _gather_rows(grp, sel, table2)
    return out.reshape(b, n * _D)


# speedup vs baseline: 1.0910x; 1.0910x over previous
import functools

import jax
import jax.numpy as jnp
from jax import lax
from jax.experimental import pallas as pl
from jax.experimental.pallas import tpu as pltpu
from jax.experimental.pallas import tpu_sc as plsc

_D = 32
_G = 4
_NC = 2
_NS = 16
_NW = _NC * _NS
_LANES = 16


def _pack_rows(t_T):
    d, rows = t_T.shape
    blk = 16384
    nblk = pl.cdiv(rows, blk)

    def body(t_ref, o_ref):
        eye = (
            lax.broadcasted_iota(jnp.int32, (d, d), 0)
            == lax.broadcasted_iota(jnp.int32, (d, d), 1)
        ).astype(jnp.float32)
        t1 = lax.dot_general(
            t_ref[...], eye, (((0,), (0,)), ((), ())),
            preferred_element_type=jnp.float32,
            precision=lax.Precision.HIGHEST,
        )
        t1r = t1.reshape(blk // _G, _G, d)
        for q in range(_G):
            o_ref[:, q * d:(q + 1) * d] = t1r[:, q, :]

    return pl.pallas_call(
        body,
        grid=(nblk,),
        in_specs=[pl.BlockSpec((d, blk), lambda g: (0, g))],
        out_specs=pl.BlockSpec((blk // _G, _G * d), lambda g: (g, 0)),
        out_shape=jax.ShapeDtypeStruct((nblk * blk // _G, _G * d), jnp.float32),
    )(t_T)


def _gather_rows(grp, sel, table2):
    total = grp.shape[0]
    per_w = total // _NW
    chunk = 160
    n_chunks = per_w // chunk

    mesh = plsc.VectorSubcoreMesh(core_axis_name="c", subcore_axis_name="s")

    @functools.partial(
        pl.kernel,
        mesh=mesh,
        out_type=jax.ShapeDtypeStruct((total, _D), jnp.float32),
        scratch_types=[
            pltpu.VMEM((per_w,), jnp.int32),
            pltpu.VMEM((per_w,), jnp.int32),
            [pltpu.VMEM((chunk, _G * _D), jnp.float32) for _ in range(2)],
            [pltpu.VMEM((chunk, _D), jnp.float32) for _ in range(2)],
            [pltpu.SemaphoreType.DMA for _ in range(2)],
            [pltpu.SemaphoreType.DMA for _ in range(2)],
        ],
        compiler_params=pltpu.CompilerParams(
            use_tc_tiling_on_sc=True, needs_layout_passes=False
        ),
    )
    def k(grp_hbm, sel_hbm, table_hbm, out_hbm, grp_v, sel_v, staged, comp,
          gsems, wsems):
        wid = lax.axis_index("s") * _NC + lax.axis_index("c")
        base = wid * per_w
        pltpu.sync_copy(grp_hbm.at[pl.ds(base, per_w)], grp_v)
        pltpu.sync_copy(sel_hbm.at[pl.ds(base, per_w)], sel_v)

        lanes_iota = lax.broadcasted_iota(jnp.int32, (_LANES,), 0)

        def gather(j, b):
            return pltpu.async_copy(
                table_hbm.at[grp_v.at[pl.ds(j * chunk, chunk)]],
                staged[b], gsems[b],
            )

        def compact(j, b):
            off = j * chunk
            for t in range(chunk):
                col0 = plsc.load_gather(
                    sel_v, [jnp.full((_LANES,), off + t, jnp.int32)]
                )
                trow = jnp.full((_LANES,), t, jnp.int32)
                for j0 in (0, _LANES):
                    val = plsc.load_gather(
                        staged[b], [trow, col0 + (j0 + lanes_iota)]
                    )
                    comp[b][t, pl.ds(j0, _LANES)] = val

        def writeout(j, b):
            return pltpu.async_copy(
                comp[b], out_hbm.at[pl.ds(base + j * chunk, chunk)], wsems[b]
            )

        npairs = n_chunks // 2

        def gather_wrapped(j, b):
            return gather(j % n_chunks, b)

        gather(0, 0)
        gather(1, 1)

        def body(p, carry):
            j = p * 2
            gwait = pltpu.make_async_copy(
                table_hbm.at[grp_v.at[pl.ds(0, chunk)]], staged[0], gsems[0]
            )
            gwait.wait()
            compact(j, 0)
            hw0 = writeout(j, 0)
            gather_wrapped(j + 2, 0)
            gwait1 = pltpu.make_async_copy(
                table_hbm.at[grp_v.at[pl.ds(0, chunk)]], staged[1], gsems[1]
            )
            gwait1.wait()
            compact(j + 1, 1)
            hw1 = writeout(j + 1, 1)
            gather_wrapped(j + 3, 1)
            hw0.wait()
            hw1.wait()
            return carry

        lax.fori_loop(0, npairs, body, 0)
        for b in (0, 1):
            pltpu.make_async_copy(
                table_hbm.at[grp_v.at[pl.ds(0, chunk)]], staged[b], gsems[b]
            ).wait()

    return k(grp, sel, table2)


def kernel(user, memory, item_table, user_table):
    b, n = memory.shape
    idx = memory.reshape(b * n).astype(jnp.int32)
    table2 = _pack_rows(item_table.T)
    grp = idx >> 2
    sel = (idx & (_G - 1)) * _D
    out = _gather_rows(grp, sel, table2)
    return out.reshape(b, n * _D)


# speedup vs baseline: 1.5939x; 1.4610x over previous
import functools

import jax
import jax.numpy as jnp
from jax import lax
from jax.experimental import pallas as pl
from jax.experimental.pallas import tpu as pltpu
from jax.experimental.pallas import tpu_sc as plsc

_D = 32
_G = 4
_NC = 2
_NS = 16
_NW = _NC * _NS
_LANES = 16


def _pack_rows(t_T):
    d, rows = t_T.shape
    blk = 16384
    nblk = pl.cdiv(rows, blk)

    def body(t_ref, o_ref):
        eye = (
            lax.broadcasted_iota(jnp.int32, (d, d), 0)
            == lax.broadcasted_iota(jnp.int32, (d, d), 1)
        ).astype(jnp.float32)
        t1 = lax.dot_general(
            t_ref[...], eye, (((0,), (0,)), ((), ())),
            preferred_element_type=jnp.float32,
        )
        t1r = t1.reshape(blk // _G, _G, d)
        for q in range(_G):
            o_ref[:, q * d:(q + 1) * d] = t1r[:, q, :]

    return pl.pallas_call(
        body,
        grid=(nblk,),
        in_specs=[pl.BlockSpec((d, blk), lambda g: (0, g))],
        out_specs=pl.BlockSpec((blk // _G, _G * d), lambda g: (g, 0)),
        out_shape=jax.ShapeDtypeStruct((nblk * blk // _G, _G * d), jnp.float32),
    )(t_T)


def _gather_rows(grp, sel, table2):
    total = grp.shape[0]
    per_w = total // _NW
    chunk = 160
    n_chunks = per_w // chunk

    mesh = plsc.VectorSubcoreMesh(core_axis_name="c", subcore_axis_name="s")

    @functools.partial(
        pl.kernel,
        mesh=mesh,
        out_type=jax.ShapeDtypeStruct((total, _D), jnp.float32),
        scratch_types=[
            pltpu.VMEM((per_w,), jnp.int32),
            pltpu.VMEM((per_w,), jnp.int32),
            [pltpu.VMEM((chunk, _G * _D), jnp.float32) for _ in range(2)],
            [pltpu.VMEM((chunk, _D), jnp.float32) for _ in range(2)],
            [pltpu.SemaphoreType.DMA for _ in range(2)],
            [pltpu.SemaphoreType.DMA for _ in range(2)],
        ],
        compiler_params=pltpu.CompilerParams(
            use_tc_tiling_on_sc=True, needs_layout_passes=False
        ),
    )
    def k(grp_hbm, sel_hbm, table_hbm, out_hbm, grp_v, sel_v, staged, comp,
          gsems, wsems):
        wid = lax.axis_index("s") * _NC + lax.axis_index("c")
        base = wid * per_w
        pltpu.sync_copy(grp_hbm.at[pl.ds(base, per_w)], grp_v)
        pltpu.sync_copy(sel_hbm.at[pl.ds(base, per_w)], sel_v)

        lanes_iota = lax.broadcasted_iota(jnp.int32, (_LANES,), 0)

        def gather(j, b):
            return pltpu.async_copy(
                table_hbm.at[grp_v.at[pl.ds(j * chunk, chunk)]],
                staged[b], gsems[b],
            )

        def compact(j, b):
            off = j * chunk
            for t in range(chunk):
                col0 = plsc.load_gather(
                    sel_v, [jnp.full((_LANES,), off + t, jnp.int32)]
                )
                trow = jnp.full((_LANES,), t, jnp.int32)
                for j0 in (0, _LANES):
                    val = plsc.load_gather(
                        staged[b], [trow, col0 + (j0 + lanes_iota)]
                    )
                    comp[b][t, pl.ds(j0, _LANES)] = val

        def writeout(j, b):
            return pltpu.async_copy(
                comp[b], out_hbm.at[pl.ds(base + j * chunk, chunk)], wsems[b]
            )

        npairs = n_chunks // 2

        def gather_wrapped(j, b):
            return gather(j % n_chunks, b)

        gather(0, 0)
        gather(1, 1)

        def body(p, carry):
            j = p * 2
            gwait = pltpu.make_async_copy(
                table_hbm.at[grp_v.at[pl.ds(0, chunk)]], staged[0], gsems[0]
            )
            gwait.wait()
            compact(j, 0)
            hw0 = writeout(j, 0)
            gather_wrapped(j + 2, 0)
            gwait1 = pltpu.make_async_copy(
                table_hbm.at[grp_v.at[pl.ds(0, chunk)]], staged[1], gsems[1]
            )
            gwait1.wait()
            compact(j + 1, 1)
            hw1 = writeout(j + 1, 1)
            gather_wrapped(j + 3, 1)
            hw0.wait()
            hw1.wait()
            return carry

        lax.fori_loop(0, npairs, body, 0)
        for b in (0, 1):
            pltpu.make_async_copy(
                table_hbm.at[grp_v.at[pl.ds(0, chunk)]], staged[b], gsems[b]
            ).wait()

    return k(grp, sel, table2)


def kernel(user, memory, item_table, user_table):
    b, n = memory.shape
    idx = memory.reshape(b * n).astype(jnp.int32)
    table2 = _pack_rows(item_table.T)
    grp = idx >> 2
    sel = (idx & (_G - 1)) * _D
    out = _gather_rows(grp, sel, table2)
    return out.reshape(b, n * _D)


# speedup vs baseline: 1.6497x; 1.0350x over previous
import functools

import jax
import jax.numpy as jnp
from jax import lax
from jax.experimental import pallas as pl
from jax.experimental.pallas import tpu as pltpu
from jax.experimental.pallas import tpu_sc as plsc

_D = 32
_G = 4
_NC = 2
_NS = 16
_NW = _NC * _NS
_LANES = 16


def _pack_rows(t_T):
    d, rows = t_T.shape
    blk = 16384
    nblk = pl.cdiv(rows, blk)

    def body(t_ref, o_ref):
        t1r = t_ref[...].T.reshape(blk // _G, _G, d)
        for q in range(_G):
            o_ref[:, q * d:(q + 1) * d] = t1r[:, q, :]

    return pl.pallas_call(
        body,
        grid=(nblk,),
        in_specs=[pl.BlockSpec((d, blk), lambda g: (0, g))],
        out_specs=pl.BlockSpec((blk // _G, _G * d), lambda g: (g, 0)),
        out_shape=jax.ShapeDtypeStruct((nblk * blk // _G, _G * d), jnp.float32),
    )(t_T)


def _gather_rows(grp, sel, table2):
    total = grp.shape[0]
    per_w = total // _NW
    chunk = 160
    n_chunks = per_w // chunk

    mesh = plsc.VectorSubcoreMesh(core_axis_name="c", subcore_axis_name="s")

    @functools.partial(
        pl.kernel,
        mesh=mesh,
        out_type=jax.ShapeDtypeStruct((total, _D), jnp.float32),
        scratch_types=[
            pltpu.VMEM((per_w,), jnp.int32),
            pltpu.VMEM((per_w,), jnp.int32),
            [pltpu.VMEM((chunk, _G * _D), jnp.float32) for _ in range(2)],
            [pltpu.VMEM((chunk, _D), jnp.float32) for _ in range(2)],
            [pltpu.SemaphoreType.DMA for _ in range(2)],
            [pltpu.SemaphoreType.DMA for _ in range(2)],
        ],
        compiler_params=pltpu.CompilerParams(
            use_tc_tiling_on_sc=True, needs_layout_passes=False
        ),
    )
    def k(grp_hbm, sel_hbm, table_hbm, out_hbm, grp_v, sel_v, staged, comp,
          gsems, wsems):
        wid = lax.axis_index("s") * _NC + lax.axis_index("c")
        base = wid * per_w
        pltpu.sync_copy(grp_hbm.at[pl.ds(base, per_w)], grp_v)
        pltpu.sync_copy(sel_hbm.at[pl.ds(base, per_w)], sel_v)

        lanes_iota = lax.broadcasted_iota(jnp.int32, (_LANES,), 0)

        def gather(j, b):
            return pltpu.async_copy(
                table_hbm.at[grp_v.at[pl.ds(j * chunk, chunk)]],
                staged[b], gsems[b],
            )

        def compact(j, b):
            off = j * chunk
            for t in range(chunk):
                col0 = plsc.load_gather(
                    sel_v, [jnp.full((_LANES,), off + t, jnp.int32)]
                )
                trow = jnp.full((_LANES,), t, jnp.int32)
                for j0 in (0, _LANES):
                    val = plsc.load_gather(
                        staged[b], [trow, col0 + (j0 + lanes_iota)]
                    )
                    comp[b][t, pl.ds(j0, _LANES)] = val

        def writeout(j, b):
            return pltpu.async_copy(
                comp[b], out_hbm.at[pl.ds(base + j * chunk, chunk)], wsems[b]
            )

        npairs = n_chunks // 2

        def gather_wrapped(j, b):
            return gather(j % n_chunks, b)

        gather(0, 0)
        gather(1, 1)

        def body(p, carry):
            j = p * 2
            gwait = pltpu.make_async_copy(
                table_hbm.at[grp_v.at[pl.ds(0, chunk)]], staged[0], gsems[0]
            )
            gwait.wait()
            compact(j, 0)
            hw0 = writeout(j, 0)
            gather_wrapped(j + 2, 0)
            gwait1 = pltpu.make_async_copy(
                table_hbm.at[grp_v.at[pl.ds(0, chunk)]], staged[1], gsems[1]
            )
            gwait1.wait()
            compact(j + 1, 1)
            hw1 = writeout(j + 1, 1)
            gather_wrapped(j + 3, 1)
            hw0.wait()
            hw1.wait()
            return carry

        lax.fori_loop(0, npairs, body, 0)
        for b in (0, 1):
            pltpu.make_async_copy(
                table_hbm.at[grp_v.at[pl.ds(0, chunk)]], staged[b], gsems[b]
            ).wait()

    return k(grp, sel, table2)


def kernel(user, memory, item_table, user_table):
    b, n = memory.shape
    idx = memory.reshape(b * n).astype(jnp.int32)
    table2 = _pack_rows(item_table.T)
    grp = idx >> 2
    sel = (idx & (_G - 1)) * _D
    out = _gather_rows(grp, sel, table2)
    return out.reshape(b, n * _D)


# speedup vs baseline: 1.8524x; 1.1229x over previous
import functools

import jax
import jax.numpy as jnp
from jax import lax
from jax.experimental import pallas as pl
from jax.experimental.pallas import tpu as pltpu
from jax.experimental.pallas import tpu_sc as plsc

_D = 32
_G = 4
_NC = 2
_NS = 16
_NW = _NC * _NS
_LANES = 16


def _pack_rows(t_T):
    d, rows = t_T.shape
    blk = 16384
    nblk = pl.cdiv(rows, blk)

    def body(t_ref, o_ref):
        t1r = t_ref[...].T.reshape(blk // _G, _G, d)
        for q in range(_G):
            o_ref[:, q * d:(q + 1) * d] = t1r[:, q, :]

    return pl.pallas_call(
        body,
        grid=(nblk,),
        in_specs=[pl.BlockSpec((d, blk), lambda g: (0, g))],
        out_specs=pl.BlockSpec((blk // _G, _G * d), lambda g: (g, 0)),
        out_shape=jax.ShapeDtypeStruct((nblk * blk // _G, _G * d), jnp.float32),
    )(t_T)


def _gather_rows(grp, sel, table2, batch, n_per_row):
    total = grp.shape[0]
    per_w = total // _NW
    chunk = 200
    n_chunks = per_w // chunk
    rows_per_chunk = chunk // n_per_row

    mesh = plsc.VectorSubcoreMesh(core_axis_name="c", subcore_axis_name="s")

    @functools.partial(
        pl.kernel,
        mesh=mesh,
        out_type=jax.ShapeDtypeStruct((batch, n_per_row * _D), jnp.float32),
        scratch_types=[
            pltpu.VMEM((per_w,), jnp.int32),
            pltpu.VMEM((per_w,), jnp.int32),
            [pltpu.VMEM((chunk, _G * _D), jnp.float32) for _ in range(2)],
            pltpu.VMEM((2 * rows_per_chunk, n_per_row * _D), jnp.float32),
            [pltpu.SemaphoreType.DMA for _ in range(2)],
            [pltpu.SemaphoreType.DMA for _ in range(2)],
        ],
        compiler_params=pltpu.CompilerParams(
            use_tc_tiling_on_sc=True, needs_layout_passes=False
        ),
    )
    def k(grp_hbm, sel_hbm, table_hbm, out_hbm, grp_v, sel_v, staged, comp,
          gsems, wsems):
        wid = lax.axis_index("s") * _NC + lax.axis_index("c")
        base = wid * per_w
        pltpu.sync_copy(grp_hbm.at[pl.ds(base, per_w)], grp_v)
        pltpu.sync_copy(sel_hbm.at[pl.ds(base, per_w)], sel_v)

        lanes_iota = lax.broadcasted_iota(jnp.int32, (_LANES,), 0)

        def gather(j, b):
            return pltpu.async_copy(
                table_hbm.at[grp_v.at[pl.ds(j * chunk, chunk)]],
                staged[b], gsems[b],
            )

        def compact(j, b, row0):
            off = j * chunk
            for t in range(chunk):
                col0 = plsc.load_gather(
                    sel_v, [jnp.full((_LANES,), off + t, jnp.int32)]
                )
                trow = jnp.full((_LANES,), t, jnp.int32)
                orow = row0 + t // n_per_row
                ocol = (t % n_per_row) * _D
                for j0 in (0, _LANES):
                    val = plsc.load_gather(
                        staged[b], [trow, col0 + (j0 + lanes_iota)]
                    )
                    comp[orow, pl.ds(ocol + j0, _LANES)] = val

        def writeout(p):
            off_rows = pl.multiple_of(
                base // n_per_row + p * 2 * rows_per_chunk,
                2 * rows_per_chunk,
            )
            return pltpu.async_copy(
                comp,
                out_hbm.at[pl.ds(off_rows, 2 * rows_per_chunk)],
                wsems[0],
            )

        npairs = n_chunks // 2

        def gather_wrapped(j, b):
            return gather(j % n_chunks, b)

        gather(0, 0)
        gather(1, 1)

        def body(p, carry):
            j = p * 2
            gwait = pltpu.make_async_copy(
                table_hbm.at[grp_v.at[pl.ds(0, chunk)]], staged[0], gsems[0]
            )
            gwait.wait()
            compact(j, 0, 0)
            gather_wrapped(j + 2, 0)
            gwait1 = pltpu.make_async_copy(
                table_hbm.at[grp_v.at[pl.ds(0, chunk)]], staged[1], gsems[1]
            )
            gwait1.wait()
            compact(j + 1, 1, rows_per_chunk)
            gather_wrapped(j + 3, 1)
            writeout(p).wait()
            return carry

        lax.fori_loop(0, npairs, body, 0)
        for b in (0, 1):
            pltpu.make_async_copy(
                table_hbm.at[grp_v.at[pl.ds(0, chunk)]], staged[b], gsems[b]
            ).wait()

    return k(grp, sel, table2)


def kernel(user, memory, item_table, user_table):
    b, n = memory.shape
    idx = memory.reshape(b * n).astype(jnp.int32)
    table2 = _pack_rows(item_table.T)
    grp = idx >> 2
    sel = (idx & (_G - 1)) * _D
    return _gather_rows(grp, sel, table2, b, n)
